# baseline (speedup 1.0000x reference)
import functools

import jax
import jax.numpy as jnp
from jax import lax
from jax.experimental import pallas as pl
from jax.experimental.pallas import tpu as pltpu
from jax.experimental.pallas import tpu_sc as plsc

_NC = 2
_NS = 16
_L = 16
_NW = _NC * _NS
_D = 1024
_ROWS = 32 * 1024
_ROWS_PER_W = _ROWS // _NW
_CHUNK = 32
_NCHUNKS = _ROWS_PER_W // _CHUNK
_GROUPS = _CHUNK // _L
_NACC = 4


def _merge(va, ka, vb, kb):
    take_b = (vb < va) | ((vb == va) & (kb < ka))
    return jnp.where(take_b, vb, va), jnp.where(take_b, kb, ka)


def _sc_argmin_body(x_hbm, o_hbm, buf, out_v, sem0, sem1):
    wid = lax.axis_index("s") * _NC + lax.axis_index("c")
    base_row = wid * _ROWS_PER_W
    base_word = base_row * _D
    chunk_words = _CHUNK * _D
    sems = (sem0, sem1)

    lane = lax.iota(jnp.int32, _L)

    descs = [None, None]

    def start(c):
        par = c % 2
        return pltpu.async_copy(
            x_hbm.at[pl.ds(base_word + c * chunk_words, chunk_words)],
            buf.at[pl.ds(par * chunk_words, chunk_words)],
            sems[par],
        )

    descs[0] = start(0)
    for c in range(_NCHUNKS):
        if c + 1 < _NCHUNKS:
            descs[(c + 1) % 2] = start(c + 1)
        descs[c % 2].wait()
        par_off = (c % 2) * chunk_words
        for g in range(_GROUPS):
            base_vec = par_off + (g * _L + lane) * _D
            cols0 = tuple(base_vec + j for j in range(_NACC))
            vals0 = tuple(jnp.full((_L,), jnp.inf, jnp.float32) for _ in range(_NACC))
            bests0 = cols0

            @plsc.parallel_loop(0, _D, step=_NACC, unroll=2,
                                carry=(vals0, bests0, cols0))
            def _loop(_, carry):
                vals, bests, cols = carry
                new_vals, new_bests, new_cols = [], [], []
                for j in range(_NACC):
                    v = plsc.load_gather(buf, [cols[j]])
                    pred = v < vals[j]
                    new_vals.append(jnp.where(pred, v, vals[j]))
                    new_bests.append(jnp.where(pred, cols[j], bests[j]))
                    new_cols.append(cols[j] + _NACC)
                return tuple(new_vals), tuple(new_bests), tuple(new_cols)

            vals, bests, _ = _loop
            v01, k01 = _merge(vals[0], bests[0], vals[1], bests[1])
            v23, k23 = _merge(vals[2], bests[2], vals[3], bests[3])
            _, kbest = _merge(v01, k01, v23, k23)
            out_v[pl.ds(c * _CHUNK + g * _L, _L)] = kbest & (_D - 1)

    pltpu.sync_copy(out_v, o_hbm.at[pl.ds(base_row, _ROWS_PER_W)])


def _sc_argmin(xf):
    mesh = plsc.VectorSubcoreMesh(core_axis_name="c", subcore_axis_name="s")
    f = pl.kernel(
        _sc_argmin_body,
        out_type=jax.ShapeDtypeStruct((_ROWS,), jnp.int32),
        mesh=mesh,
        compiler_params=pltpu.CompilerParams(needs_layout_passes=False),
        scratch_types=[
            pltpu.VMEM((2 * _CHUNK * _D,), jnp.float32),
            pltpu.VMEM((_ROWS_PER_W,), jnp.int32),
            pltpu.SemaphoreType.DMA,
            pltpu.SemaphoreType.DMA,
        ],
    )
    return f(xf)


def kernel(x):
    b, d1, d2 = x.shape
    xf = x.reshape(b * d1 * d2)
    return _sc_argmin(xf).reshape(b, d1)


# speedup vs baseline: 2.2128x; 2.2128x over previous
import jax
import jax.numpy as jnp
from jax import lax
from jax.experimental import pallas as pl
from jax.experimental.pallas import tpu as pltpu
from jax.experimental.pallas import tpu_sc as plsc

_NC = 2
_NS = 16
_L = 16
_NW = _NC * _NS
_D = 1024
_SLICES = _D // _L
_ROWS = 32 * 1024
_ROWS_PER_W = _ROWS // _NW
_CHUNK = 32
_NCHUNKS = _ROWS_PER_W // _CHUNK
_NACC = 4


def _merge(va, sa, vb, sb):
    take_b = (vb < va) | ((vb == va) & (sb < sa))
    return jnp.where(take_b, vb, va), jnp.where(take_b, sb, sa)


def _sc_argmin_body(x_hbm, o_hbm, buf, out_v, sem0, sem1):
    wid = lax.axis_index("s") * _NC + lax.axis_index("c")
    base_row = wid * _ROWS_PER_W
    base_word = base_row * _D
    chunk_words = _CHUNK * _D
    sems = (sem0, sem1)

    lane = lax.iota(jnp.int32, _L)
    inf16 = jnp.full((_L,), jnp.inf, jnp.float32)
    zero16 = jnp.zeros((_L,), jnp.int32)

    def copy_chunk(c, par):
        return pltpu.make_async_copy(
            x_hbm.at[pl.ds(base_word + c * chunk_words, chunk_words)],
            buf.at[pl.ds(par * chunk_words, chunk_words)],
            sems[par],
        )

    copy_chunk(0, 0).start()
    copy_chunk(1, 1).start()

    def do_chunk(c, par):
        copy_chunk(c, par).wait()
        par_off = par * chunk_words

        def row_body(r, res_vec):
            row_off = par_off + r * _D

            @plsc.parallel_loop(
                0, _SLICES, step=_NACC,
                carry=((inf16,) * _NACC, (zero16,) * _NACC),
            )
            def _loop(s, carry):
                vals, bests = carry
                new_vals, new_bests = [], []
                for j in range(_NACC):
                    v = buf[pl.ds(row_off + (s + j) * _L, _L)]
                    pred = v < vals[j]
                    new_vals.append(jnp.where(pred, v, vals[j]))
                    new_bests.append(
                        jnp.where(pred, jnp.full((_L,), s + j, jnp.int32),
                                  bests[j]))
                return tuple(new_vals), tuple(new_bests)

            vals, bests = _loop
            v01, s01 = _merge(vals[0], bests[0], vals[1], bests[1])
            v23, s23 = _merge(vals[2], bests[2], vals[3], bests[3])
            vm, sm = _merge(v01, s01, v23, s23)
            idx16 = sm * _L + lane
            m = lax.reduce_min(vm, (0,))
            idxc = jnp.where(vm == m, idx16, _D)
            best = lax.reduce_min(idxc, (0,))
            res_vec = jnp.where(lane == (r & (_L - 1)),
                                jnp.full((_L,), best, jnp.int32), res_vec)

            @pl.when((r & (_L - 1)) == _L - 1)
            def _():
                out_v[pl.ds(c * _CHUNK + (r & ~(_L - 1)), _L)] = res_vec

            return res_vec

        lax.fori_loop(0, _CHUNK, row_body, zero16)

        @pl.when(c + 2 < _NCHUNKS)
        def _():
            copy_chunk(c + 2, par).start()

    def pair_body(p, carry):
        do_chunk(2 * p, 0)
        do_chunk(2 * p + 1, 1)
        return carry

    lax.fori_loop(0, _NCHUNKS // 2, pair_body, 0)
    pltpu.sync_copy(out_v, o_hbm.at[pl.ds(base_row, _ROWS_PER_W)])


def _sc_argmin(xf):
    mesh = plsc.VectorSubcoreMesh(core_axis_name="c", subcore_axis_name="s")
    f = pl.kernel(
        _sc_argmin_body,
        out_type=jax.ShapeDtypeStruct((_ROWS,), jnp.int32),
        mesh=mesh,
        compiler_params=pltpu.CompilerParams(needs_layout_passes=False),
        scratch_types=[
            pltpu.VMEM((2 * _CHUNK * _D,), jnp.float32),
            pltpu.VMEM((_ROWS_PER_W,), jnp.int32),
            pltpu.SemaphoreType.DMA,
            pltpu.SemaphoreType.DMA,
        ],
    )
    return f(xf)


def kernel(x):
    b, d1, d2 = x.shape
    xf = x.reshape(b * d1 * d2)
    return _sc_argmin(xf).reshape(b, d1)


# speedup vs baseline: 2.8231x; 1.2758x over previous
import jax
import jax.numpy as jnp
from jax import lax
from jax.experimental import pallas as pl
from jax.experimental.pallas import tpu as pltpu
from jax.experimental.pallas import tpu_sc as plsc

_NC = 2
_NS = 16
_L = 16
_NW = _NC * _NS
_D = 1024
_SLICES = _D // _L
_ROWS = 32 * 1024
_ROWS_PER_W = _ROWS // _NW
_CHUNK = 32
_NCHUNKS = _ROWS_PER_W // _CHUNK
_NACC = 4


def _merge(va, sa, vb, sb):
    take_b = (vb < va) | ((vb == va) & (sb < sa))
    return jnp.where(take_b, vb, va), jnp.where(take_b, sb, sa)


def _sc_argmin_body(x_hbm, o_hbm, buf, out_v, sem0, sem1):
    wid = lax.axis_index("s") * _NC + lax.axis_index("c")
    base_row = wid * _ROWS_PER_W
    base_word = base_row * _D
    chunk_words = _CHUNK * _D
    sems = (sem0, sem1)

    lane = lax.iota(jnp.int32, _L)
    inf16 = jnp.full((_L,), jnp.inf, jnp.float32)
    zero16 = jnp.zeros((_L,), jnp.int32)

    def copy_chunk(c, par):
        return pltpu.make_async_copy(
            x_hbm.at[pl.ds(base_word + c * chunk_words, chunk_words)],
            buf.at[pl.ds(par * chunk_words, chunk_words)],
            sems[par],
        )

    copy_chunk(0, 0).start()
    copy_chunk(1, 1).start()

    def do_chunk(c, par):
        copy_chunk(c, par).wait()
        par_off = par * chunk_words

        def row_body(r, res_vec):
            row_off = par_off + r * _D

            @plsc.parallel_loop(
                0, _SLICES, step=_NACC, unroll=4,
                carry=((inf16,) * _NACC, (zero16,) * _NACC),
            )
            def _loop(s, carry):
                vals, bests = carry
                new_vals, new_bests = [], []
                for j in range(_NACC):
                    v = buf[pl.ds(row_off + (s + j) * _L, _L)]
                    pred = v < vals[j]
                    new_vals.append(jnp.where(pred, v, vals[j]))
                    new_bests.append(
                        jnp.where(pred, jnp.full((_L,), s + j, jnp.int32),
                                  bests[j]))
                return tuple(new_vals), tuple(new_bests)

            vals, bests = _loop
            v01, s01 = _merge(vals[0], bests[0], vals[1], bests[1])
            v23, s23 = _merge(vals[2], bests[2], vals[3], bests[3])
            vm, sm = _merge(v01, s01, v23, s23)
            idx16 = sm * _L + lane
            m = lax.reduce_min(vm, (0,))
            idxc = jnp.where(vm == m, idx16, _D)
            best = lax.reduce_min(idxc, (0,))
            res_vec = jnp.where(lane == (r & (_L - 1)),
                                jnp.full((_L,), best, jnp.int32), res_vec)

            @pl.when((r & (_L - 1)) == _L - 1)
            def _():
                out_v[pl.ds(c * _CHUNK + (r & ~(_L - 1)), _L)] = res_vec

            return res_vec

        lax.fori_loop(0, _CHUNK, row_body, zero16)

        @pl.when(c + 2 < _NCHUNKS)
        def _():
            copy_chunk(c + 2, par).start()

    def pair_body(p, carry):
        do_chunk(2 * p, 0)
        do_chunk(2 * p + 1, 1)
        return carry

    lax.fori_loop(0, _NCHUNKS // 2, pair_body, 0)
    pltpu.sync_copy(out_v, o_hbm.at[pl.ds(base_row, _ROWS_PER_W)])


def _sc_argmin(xf):
    mesh = plsc.VectorSubcoreMesh(core_axis_name="c", subcore_axis_name="s")
    f = pl.kernel(
        _sc_argmin_body,
        out_type=jax.ShapeDtypeStruct((_ROWS,), jnp.int32),
        mesh=mesh,
        compiler_params=pltpu.CompilerParams(needs_layout_passes=False),
        scratch_types=[
            pltpu.VMEM((2 * _CHUNK * _D,), jnp.float32),
            pltpu.VMEM((_ROWS_PER_W,), jnp.int32),
            pltpu.SemaphoreType.DMA,
            pltpu.SemaphoreType.DMA,
        ],
    )
    return f(xf)


def kernel(x):
    b, d1, d2 = x.shape
    xf = x.reshape(b * d1 * d2)
    return _sc_argmin(xf).reshape(b, d1)
